# baseline (speedup 1.0000x reference)
import functools

import jax
import jax.numpy as jnp
from jax import lax
from jax.experimental import pallas as pl
from jax.experimental.pallas import tpu as pltpu
from jax.experimental.pallas import tpu_sc as plsc

VOCAB = 1000000
D = 64
MAXLEN = 200
VDIM = 128
B = 1024
T = 200

NC = 2
NS = 16
NW = NC * NS
SEQ_PER_W = B // NW
SEQ_PER_G = 4
N_GROUPS = SEQ_PER_W // SEQ_PER_G
S0, S1 = 128, 72


def _sc_gather_body(table_hbm, idx_hbm, out_hbm, idx_v, rows_v, sem):
    wid = lax.axis_index("s") * NC + lax.axis_index("c")
    base = wid * SEQ_PER_W
    pltpu.sync_copy(idx_hbm.at[pl.ds(base, SEQ_PER_W)], idx_v)

    @pl.loop(0, N_GROUPS)
    def _group(g):
        copies = []
        for k in range(SEQ_PER_G):
            s = g * SEQ_PER_G + k
            copies.append(pltpu.async_copy(
                table_hbm.at[idx_v.at[s, pl.ds(0, S0)]],
                rows_v.at[k, pl.ds(0, S0)], sem))
            copies.append(pltpu.async_copy(
                table_hbm.at[idx_v.at[s, pl.ds(S0, S1)]],
                rows_v.at[k, pl.ds(S0, S1)], sem))
        for c in copies:
            c.wait()
        pltpu.sync_copy(rows_v, out_hbm.at[pl.ds(base + g * SEQ_PER_G, SEQ_PER_G)])


def _sc_gather(table, idx):
    mesh = plsc.VectorSubcoreMesh(core_axis_name="c", subcore_axis_name="s")
    return pl.kernel(
        _sc_gather_body,
        out_type=jax.ShapeDtypeStruct((B, T, D), jnp.float32),
        mesh=mesh,
        scratch_types=[
            pltpu.VMEM((SEQ_PER_W, T), jnp.int32),
            pltpu.VMEM((SEQ_PER_G, T, D), jnp.float32),
            pltpu.SemaphoreType.DMA,
        ],
        compiler_params=pltpu.CompilerParams(use_tc_tiling_on_sc=False),
    )(table, idx)


BB = 8


def _tc_body(g_ref, vis_ref, pos_ref, w_ref, gamma_ref, beta_ref, out_ref):
    x = g_ref[...] + pos_ref[...]
    v = vis_ref[...].reshape(BB * T, VDIM)
    proj = jnp.dot(v, w_ref[...], preferred_element_type=jnp.float32)
    x = x + proj.reshape(BB, T, D)
    mean = jnp.mean(x, axis=-1, keepdims=True)
    xc = x - mean
    var = jnp.mean(xc * xc, axis=-1, keepdims=True)
    out_ref[...] = xc * lax.rsqrt(var + 1e-6) * gamma_ref[...] + beta_ref[...]


def _tc_fused(gathered, vis, pos, w_t, gamma, beta):
    return pl.pallas_call(
        _tc_body,
        grid=(B // BB,),
        in_specs=[
            pl.BlockSpec((BB, T, D), lambda i: (i, 0, 0)),
            pl.BlockSpec((BB, T, VDIM), lambda i: (i, 0, 0)),
            pl.BlockSpec((1, T, D), lambda i: (0, 0, 0)),
            pl.BlockSpec((VDIM, D), lambda i: (0, 0)),
            pl.BlockSpec((1, 1, D), lambda i: (0, 0, 0)),
            pl.BlockSpec((1, 1, D), lambda i: (0, 0, 0)),
        ],
        out_specs=pl.BlockSpec((BB, T, D), lambda i: (i, 0, 0)),
        out_shape=jax.ShapeDtypeStruct((B, T, D), jnp.float32),
    )(gathered, vis, pos, w_t, gamma, beta)


def kernel(seq, visual_features, token_table, pos_table, W_visual, ln_gamma, ln_beta):
    idx = seq.astype(jnp.int32)
    gathered = _sc_gather(token_table, idx)
    out = _tc_fused(
        gathered,
        visual_features,
        pos_table[None],
        W_visual.T,
        ln_gamma.reshape(1, 1, D),
        ln_beta.reshape(1, 1, D),
    )
    return out


# speedup vs baseline: 1.0935x; 1.0935x over previous
import jax
import jax.numpy as jnp
from jax import lax
from jax.experimental import pallas as pl
from jax.experimental.pallas import tpu as pltpu
from jax.experimental.pallas import tpu_sc as plsc

VOCAB = 1000000
D = 64
MAXLEN = 200
VDIM = 128
B = 1024
T = 200

NC = 2
NS = 16
NW = NC * NS
PER_W = B * T // NW


N_STREAMS = 50
STREAM = 128
GROUP_STREAMS = 5
GROUP = GROUP_STREAMS * STREAM
N_GROUPS = PER_W // GROUP


def _sc_gather_body(table_hbm, idx_hbm, out_hbm, idx_v, rows_v, sem):
    wid = lax.axis_index("s") * NC + lax.axis_index("c")
    base = wid * PER_W
    pltpu.sync_copy(idx_hbm.at[wid], idx_v)

    @pl.loop(0, N_GROUPS)
    def _group(g):
        copies = []
        for j in range(GROUP_STREAMS):
            copies.append(pltpu.async_copy(
                table_hbm.at[idx_v.at[g * GROUP_STREAMS + j]],
                rows_v.at[pl.ds(j * STREAM, STREAM)],
                sem,
            ))
        for c in copies:
            c.wait()
        pltpu.sync_copy(rows_v, out_hbm.at[pl.ds(base + g * GROUP, GROUP)])


def _sc_gather(table, idx):
    mesh = plsc.VectorSubcoreMesh(core_axis_name="c", subcore_axis_name="s")
    return pl.kernel(
        _sc_gather_body,
        out_type=jax.ShapeDtypeStruct((B * T, D), jnp.float32),
        mesh=mesh,
        scratch_types=[
            pltpu.VMEM((N_STREAMS, STREAM), jnp.int32),
            pltpu.VMEM((GROUP, D), jnp.float32),
            pltpu.SemaphoreType.DMA,
        ],
        compiler_params=pltpu.CompilerParams(use_tc_tiling_on_sc=False),
    )(table, idx)


TBLK = 8


def _mm_body(vis_ref, w_ref, pos_ref, tmp_ref):
    for t in range(TBLK):
        v = vis_ref[:, t, :]
        p = lax.dot_general(
            w_ref[...], v, (((1,), (1,)), ((), ())),
            preferred_element_type=jnp.float32,
        )
        tmp_ref[t] = p + pos_ref[t][:, None]


def _tc_matmul(vis, w, pos):
    return pl.pallas_call(
        _mm_body,
        grid=(T // TBLK,),
        in_specs=[
            pl.BlockSpec((B, TBLK, VDIM), lambda i: (0, i, 0)),
            pl.BlockSpec((D, VDIM), lambda i: (0, 0)),
            pl.BlockSpec((TBLK, D), lambda i: (i, 0)),
        ],
        out_specs=pl.BlockSpec((TBLK, D, B), lambda i: (i, 0, 0)),
        out_shape=jax.ShapeDtypeStruct((T, D, B), jnp.float32),
    )(vis, w, pos)


def _fin_body(g_ref, tmp_ref, gamma_ref, beta_ref, out_ref):
    x = g_ref[...] + tmp_ref[...]
    mean = jnp.mean(x, axis=1, keepdims=True)
    xc = x - mean
    var = jnp.mean(xc * xc, axis=1, keepdims=True)
    out_ref[...] = xc * lax.rsqrt(var + 1e-6) * gamma_ref[...] + beta_ref[...]


def _tc_final(g_t, tmp, gamma, beta):
    return pl.pallas_call(
        _fin_body,
        grid=(T // TBLK,),
        in_specs=[
            pl.BlockSpec((TBLK, D, B), lambda i: (i, 0, 0)),
            pl.BlockSpec((TBLK, D, B), lambda i: (i, 0, 0)),
            pl.BlockSpec((1, D, 1), lambda i: (0, 0, 0)),
            pl.BlockSpec((1, D, 1), lambda i: (0, 0, 0)),
        ],
        out_specs=pl.BlockSpec((TBLK, D, B), lambda i: (i, 0, 0)),
        out_shape=jax.ShapeDtypeStruct((T, D, B), jnp.float32),
    )(g_t, tmp, gamma, beta)


def kernel(seq, visual_features, token_table, pos_table, W_visual, ln_gamma, ln_beta):
    idx = seq.astype(jnp.int32).reshape(NW, N_STREAMS, STREAM)
    gathered = _sc_gather(token_table, idx).reshape(B, T, D)
    tmp = _tc_matmul(visual_features, W_visual, pos_table)
    g_t = jnp.transpose(gathered, (1, 2, 0))
    out_t = _tc_final(
        g_t, tmp, ln_gamma.reshape(1, D, 1), ln_beta.reshape(1, D, 1)
    )
    return jnp.transpose(out_t, (2, 0, 1))
